# baseline (speedup 1.0000x reference)
import functools

import jax
import jax.numpy as jnp
from jax import lax
from jax.experimental import pallas as pl
from jax.experimental.pallas import tpu as pltpu
from jax.experimental.pallas import tpu_sc as plsc

_B, _N, _CIN = 4, 8192, 32
_S = 1024
_K = 32
_R = 0.2
_EPS = 1e-5
_M = _B * _S * _K
_DTAB = 128


def _fps_body(xyzr_ref, xg_ref, cent_ref, nxyz_ref):
    rows = lax.broadcasted_iota(jnp.int32, (64, 128), 0)
    cols = lax.broadcasted_iota(jnp.int32, (64, 128), 1)
    flat = rows * 128 + cols
    crow = lax.broadcasted_iota(jnp.int32, (8, 128), 0)
    ccol = lax.broadcasted_iota(jnp.int32, (8, 128), 1)
    cflat = crow * 128 + ccol
    X = [xg_ref[b, 0] for b in range(_B)]
    Y = [xg_ref[b, 1] for b in range(_B)]
    Z = [xg_ref[b, 2] for b in range(_B)]

    def body(i, state):
        cents, dists, cxs, cys, czs, fars = state
        out = ([], [], [], [], [], [])
        for b in range(_B):
            cent = jnp.where(cflat == i, fars[b], cents[b])
            nxyz_ref[b, pl.ds(i, 1), :] = jnp.concatenate(
                [cxs[b], cys[b], czs[b]], axis=1)
            dx = X[b] - cxs[b]
            dy = Y[b] - cys[b]
            dz = Z[b] - czs[b]
            d = dx * dx + dy * dy + dz * dz
            dist = jnp.minimum(dists[b], d)
            m = jnp.max(dist, axis=(0, 1), keepdims=True)
            sel = dist == m
            far2 = jnp.min(jnp.where(sel, flat, jnp.int32(_N)),
                           axis=(0, 1), keepdims=True)
            cx2 = jnp.max(jnp.where(sel, X[b], -1e30), axis=(0, 1), keepdims=True)
            cy2 = jnp.max(jnp.where(sel, Y[b], -1e30), axis=(0, 1), keepdims=True)
            cz2 = jnp.max(jnp.where(sel, Z[b], -1e30), axis=(0, 1), keepdims=True)
            for lst, v in zip(out, (cent, dist, cx2, cy2, cz2, far2)):
                lst.append(v)
        return tuple(tuple(l) for l in out)

    init = (tuple(jnp.zeros((8, 128), jnp.int32) for _ in range(_B)),
            tuple(jnp.full((64, 128), 1e10, jnp.float32) for _ in range(_B)),
            tuple(xyzr_ref[b, 0:1, 0:1] for b in range(_B)),
            tuple(xyzr_ref[b, 0:1, 1:2] for b in range(_B)),
            tuple(xyzr_ref[b, 0:1, 2:3] for b in range(_B)),
            tuple(jnp.zeros((1, 1), jnp.int32) for _ in range(_B)))
    cents, _, _, _, _, _ = lax.fori_loop(0, _S, body, init)
    for b in range(_B):
        cent_ref[b] = cents[b]


def _run_fps(xyz):
    xg = jnp.transpose(xyz, (0, 2, 1)).reshape(_B, 3, 64, 128)
    cent, nxyz = pl.pallas_call(
        _fps_body,
        grid=(1,),
        in_specs=[
            pl.BlockSpec((_B, _N, 3), lambda i: (0, 0, 0)),
            pl.BlockSpec((_B, 3, 64, 128), lambda i: (0, 0, 0, 0)),
        ],
        out_specs=[
            pl.BlockSpec((_B, 8, 128), lambda i: (0, 0, 0)),
            pl.BlockSpec((_B, _S, 3), lambda i: (0, 0, 0)),
        ],
        out_shape=[
            jax.ShapeDtypeStruct((_B, 8, 128), jnp.int32),
            jax.ShapeDtypeStruct((_B, _S, 3), jnp.float32),
        ],
    )(xyz, xg)
    return cent, nxyz


_SB = 128


def _bq_body(xt_ref, nx_ref, idx_ref):
    b = pl.program_id(0)
    xt = xt_ref[0]
    nx = nx_ref[0, 0]
    xsq = jnp.sum(xt * xt, axis=0, keepdims=True)
    nsq = jnp.sum(nx * nx, axis=1, keepdims=True)
    dot = jnp.dot(nx, xt, preferred_element_type=jnp.float32)
    sqr = nsq + xsq - 2.0 * dot
    mask = jnp.logical_not(sqr > _R ** 2)
    rank = mask.astype(jnp.int32)
    sh = 1
    while sh < _N:
        z = jnp.zeros((_SB, sh), jnp.int32)
        rank = rank + jnp.concatenate([z, rank[:, :-sh]], axis=1)
        sh *= 2
    cnt = []
    for k in range(_K):
        c = jnp.sum((rank <= k).astype(jnp.int32), axis=1, keepdims=True)
        cnt.append(c)
    Yc = jnp.concatenate(cnt, axis=1)
    Yc = jnp.where(Yc == _N, Yc[:, 0:1], Yc)
    idx_ref[0, 0] = Yc + b * _N


def _run_ball_query(xyz, nxyz):
    xt = jnp.transpose(xyz, (0, 2, 1))
    nblk = _S // _SB
    idx = pl.pallas_call(
        _bq_body,
        grid=(_B, nblk),
        in_specs=[
            pl.BlockSpec((1, 3, _N), lambda b, j: (b, 0, 0)),
            pl.BlockSpec((1, 1, _SB, 3), lambda b, j: (b, j, 0, 0)),
        ],
        out_specs=pl.BlockSpec((1, 1, _SB, _K), lambda b, j: (b, j, 0, 0)),
        out_shape=jax.ShapeDtypeStruct((_B, nblk, _SB, _K), jnp.int32),
    )(xt, nxyz.reshape(_B, nblk, _SB, 3))
    return idx.reshape(_M)


_NW = 32
_RPW = _M // _NW
_CH = 512


def _sc_gather_body(table_hbm, idx_hbm, out_hbm, idx_v, rows_v, sem):
    wid = lax.axis_index("s") * 2 + lax.axis_index("c")
    base = wid * _RPW

    def chunk(c, carry):
        off = base + c * _CH
        pltpu.sync_copy(idx_hbm.at[pl.ds(off, _CH)], idx_v)
        pltpu.async_copy(table_hbm.at[idx_v], rows_v, sem).wait()
        pltpu.sync_copy(rows_v, out_hbm.at[pl.ds(off, _CH)])
        return carry

    lax.fori_loop(0, _RPW // _CH, chunk, 0)


def _run_gather(table, idx):
    mesh = plsc.VectorSubcoreMesh(core_axis_name="c", subcore_axis_name="s")
    fn = functools.partial(
        pl.kernel,
        mesh=mesh,
        out_type=jax.ShapeDtypeStruct((_M, _DTAB), jnp.float32),
        scratch_types=[
            pltpu.VMEM((_CH,), jnp.int32),
            pltpu.VMEM((_CH, _DTAB), jnp.float32),
            pltpu.SemaphoreType.DMA,
        ],
    )(_sc_gather_body)
    return fn(table, idx)


_RB = 2048


def _mlp0_body(g_ref, c_ref, w_ref, b_ref, y_ref, st_ref, acc):
    i = pl.program_id(0)
    g = g_ref[...]
    cx = c_ref[...]
    dp = (g[:, 0:3] - cx) / _R
    x0 = jnp.concatenate([dp, g[:, 3:_DTAB]], axis=1)
    y = jnp.dot(x0, w_ref[...], preferred_element_type=jnp.float32)
    y = y + b_ref[...]
    s = jnp.sum(y, axis=0, keepdims=True)
    sq = jnp.sum(y * y, axis=0, keepdims=True)
    @pl.when(i == 0)
    def _():
        acc[0:1, :] = s
        acc[1:2, :] = sq
    @pl.when(i > 0)
    def _():
        acc[0:1, :] = acc[0:1, :] + s
        acc[1:2, :] = acc[1:2, :] + sq
    y_ref[...] = y
    @pl.when(i == pl.num_programs(0) - 1)
    def _():
        st_ref[...] = acc[...]


def _mlp_mid_body(y_ref, st_in_ref, ga_ref, be_ref, w_ref, b_ref,
                  y_out_ref, st_ref, acc):
    i = pl.program_id(0)
    mean = st_in_ref[0:1, :] / _M
    var = st_in_ref[1:2, :] / _M - mean * mean
    scale = ga_ref[...] / jnp.sqrt(var + _EPS)
    shift = be_ref[...] - mean * scale
    h = jnp.maximum(y_ref[...] * scale + shift, 0.0)
    y = jnp.dot(h, w_ref[...], preferred_element_type=jnp.float32)
    y = y + b_ref[...]
    s = jnp.sum(y, axis=0, keepdims=True)
    sq = jnp.sum(y * y, axis=0, keepdims=True)
    @pl.when(i == 0)
    def _():
        acc[0:1, :] = s
        acc[1:2, :] = sq
    @pl.when(i > 0)
    def _():
        acc[0:1, :] = acc[0:1, :] + s
        acc[1:2, :] = acc[1:2, :] + sq
    y_out_ref[...] = y
    @pl.when(i == pl.num_programs(0) - 1)
    def _():
        st_ref[...] = acc[...]


def _mlp_final_body(y_ref, st_in_ref, ga_ref, be_ref, out_ref):
    mean = st_in_ref[0:1, :] / _M
    var = st_in_ref[1:2, :] / _M - mean * mean
    scale = ga_ref[...] / jnp.sqrt(var + _EPS)
    shift = be_ref[...] - mean * scale
    h = jnp.maximum(y_ref[...] * scale + shift, 0.0)
    h3 = h.reshape(_RB // _K, _K, h.shape[1])
    out_ref[...] = jnp.max(h3, axis=1)


def _run_mlp(g, cexp, params):
    (w0, b0, g0, be0), (w1, b1, g1, be1), (w2, b2, g2, be2) = params
    nsteps = _M // _RB
    w0p = jnp.zeros((_DTAB, 32), jnp.float32).at[0:35, :].set(w0.T)

    y0, st0 = pl.pallas_call(
        _mlp0_body,
        grid=(nsteps,),
        in_specs=[
            pl.BlockSpec((_RB, _DTAB), lambda i: (i, 0)),
            pl.BlockSpec((_RB, 3), lambda i: (i, 0)),
            pl.BlockSpec((_DTAB, 32), lambda i: (0, 0)),
            pl.BlockSpec((1, 32), lambda i: (0, 0)),
        ],
        out_specs=[
            pl.BlockSpec((_RB, 32), lambda i: (i, 0)),
            pl.BlockSpec((2, 32), lambda i: (0, 0)),
        ],
        out_shape=[
            jax.ShapeDtypeStruct((_M, 32), jnp.float32),
            jax.ShapeDtypeStruct((2, 32), jnp.float32),
        ],
        scratch_shapes=[pltpu.VMEM((2, 32), jnp.float32)],
    )(g, cexp, w0p, b0.reshape(1, 32))

    def mid(y_in, st_in, ga, be, w, b, cout):
        return pl.pallas_call(
            _mlp_mid_body,
            grid=(nsteps,),
            in_specs=[
                pl.BlockSpec((_RB, y_in.shape[1]), lambda i: (i, 0)),
                pl.BlockSpec((2, y_in.shape[1]), lambda i: (0, 0)),
                pl.BlockSpec((1, y_in.shape[1]), lambda i: (0, 0)),
                pl.BlockSpec((1, y_in.shape[1]), lambda i: (0, 0)),
                pl.BlockSpec((y_in.shape[1], cout), lambda i: (0, 0)),
                pl.BlockSpec((1, cout), lambda i: (0, 0)),
            ],
            out_specs=[
                pl.BlockSpec((_RB, cout), lambda i: (i, 0)),
                pl.BlockSpec((2, cout), lambda i: (0, 0)),
            ],
            out_shape=[
                jax.ShapeDtypeStruct((_M, cout), jnp.float32),
                jax.ShapeDtypeStruct((2, cout), jnp.float32),
            ],
            scratch_shapes=[pltpu.VMEM((2, cout), jnp.float32)],
        )(y_in, st_in, ga.reshape(1, -1), be.reshape(1, -1), w.T, b.reshape(1, -1))

    y1, st1 = mid(y0, st0, g0, be0, w1, b1, 32)
    y2, st2 = mid(y1, st1, g1, be1, w2, b2, 64)

    out = pl.pallas_call(
        _mlp_final_body,
        grid=(nsteps,),
        in_specs=[
            pl.BlockSpec((_RB, 64), lambda i: (i, 0)),
            pl.BlockSpec((2, 64), lambda i: (0, 0)),
            pl.BlockSpec((1, 64), lambda i: (0, 0)),
            pl.BlockSpec((1, 64), lambda i: (0, 0)),
        ],
        out_specs=pl.BlockSpec((_RB // _K, 64), lambda i: (i, 0)),
        out_shape=jax.ShapeDtypeStruct((_M // _K, 64), jnp.float32),
    )(y2, st2, g2.reshape(1, 64), be2.reshape(1, 64))
    return out


def kernel(xyz, feats, W0, b0, gamma0, beta0, W1, b1, gamma1, beta1,
           W2, b2, gamma2, beta2):
    cent, new_xyz = _run_fps(xyz)
    idx = _run_ball_query(xyz, new_xyz)
    table = jnp.zeros((_B * _N, _DTAB), jnp.float32)
    table = table.at[:, 0:3].set(xyz.reshape(_B * _N, 3))
    table = table.at[:, 3:3 + _CIN].set(feats.reshape(_B * _N, _CIN))
    g = _run_gather(table, idx)
    cexp = jnp.broadcast_to(new_xyz[:, :, None, :], (_B, _S, _K, 3)).reshape(_M, 3)
    params = [(W0, b0, gamma0, beta0), (W1, b1, gamma1, beta1),
              (W2, b2, gamma2, beta2)]
    new_feats = _run_mlp(g, cexp, params).reshape(_B, _S, 64)
    return (new_xyz, new_feats)


# speedup vs baseline: 1.6358x; 1.6358x over previous
import functools

import jax
import jax.numpy as jnp
from jax import lax
from jax.experimental import pallas as pl
from jax.experimental.pallas import tpu as pltpu
from jax.experimental.pallas import tpu_sc as plsc

_B, _N, _CIN = 4, 8192, 32
_S = 1024
_K = 32
_R = 0.2
_EPS = 1e-5
_M = _B * _S * _K
_DTAB = 128


def _fps_body(xyzr_ref, xg_ref, cent_ref, nxyz_ref):
    flat = (lax.broadcasted_iota(jnp.int32, (_B, 64, 128), 1) * 128
            + lax.broadcasted_iota(jnp.int32, (_B, 64, 128), 2))
    cflat = (lax.broadcasted_iota(jnp.int32, (_B, 8, 128), 1) * 128
             + lax.broadcasted_iota(jnp.int32, (_B, 8, 128), 2))
    X = xg_ref[:, 0]
    Y = xg_ref[:, 1]
    Z = xg_ref[:, 2]

    def body(i, state):
        cent, dist, cxs, cys, czs, fars = state
        cent = jnp.where(cflat == i, fars, cent)
        nxyz_ref[:, pl.ds(i, 1), :] = jnp.concatenate([cxs, cys, czs], axis=2)
        dx = X - cxs
        dy = Y - cys
        dz = Z - czs
        d = dx * dx + dy * dy + dz * dz
        dist = jnp.minimum(dist, d)
        m = jnp.max(dist, axis=(1, 2), keepdims=True)
        sel = dist == m
        far2 = jnp.min(jnp.where(sel, flat, jnp.int32(_N)),
                       axis=(1, 2), keepdims=True)
        cx2 = jnp.max(jnp.where(sel, X, -1e30), axis=(1, 2), keepdims=True)
        cy2 = jnp.max(jnp.where(sel, Y, -1e30), axis=(1, 2), keepdims=True)
        cz2 = jnp.max(jnp.where(sel, Z, -1e30), axis=(1, 2), keepdims=True)
        return (cent, dist, cx2, cy2, cz2, far2)

    init = (jnp.zeros((_B, 8, 128), jnp.int32),
            jnp.full((_B, 64, 128), 1e10, jnp.float32),
            xyzr_ref[:, 0:1, 0:1],
            xyzr_ref[:, 0:1, 1:2],
            xyzr_ref[:, 0:1, 2:3],
            jnp.zeros((_B, 1, 1), jnp.int32))
    cents, _, _, _, _, _ = lax.fori_loop(0, _S, body, init)
    cent_ref[...] = cents


def _run_fps(xyz):
    xg = jnp.transpose(xyz, (0, 2, 1)).reshape(_B, 3, 64, 128)
    cent, nxyz = pl.pallas_call(
        _fps_body,
        grid=(1,),
        in_specs=[
            pl.BlockSpec((_B, _N, 3), lambda i: (0, 0, 0)),
            pl.BlockSpec((_B, 3, 64, 128), lambda i: (0, 0, 0, 0)),
        ],
        out_specs=[
            pl.BlockSpec((_B, 8, 128), lambda i: (0, 0, 0)),
            pl.BlockSpec((_B, _S, 3), lambda i: (0, 0, 0)),
        ],
        out_shape=[
            jax.ShapeDtypeStruct((_B, 8, 128), jnp.int32),
            jax.ShapeDtypeStruct((_B, _S, 3), jnp.float32),
        ],
    )(xyz, xg)
    return cent, nxyz


_SB = 128


def _bq_body(xt_ref, nx_ref, idx_ref):
    b = pl.program_id(0)
    xt = xt_ref[0]
    nx = nx_ref[0, 0]
    xsq = jnp.sum(xt * xt, axis=0, keepdims=True)
    nsq = jnp.sum(nx * nx, axis=1, keepdims=True)
    dot = jnp.dot(nx, xt, preferred_element_type=jnp.float32)
    sqr = nsq + xsq - 2.0 * dot
    mask = jnp.logical_not(sqr > _R ** 2)
    rank = mask.astype(jnp.int32)
    sh = 1
    while sh < _N:
        z = jnp.zeros((_SB, sh), jnp.int32)
        rank = rank + jnp.concatenate([z, rank[:, :-sh]], axis=1)
        sh *= 2
    cnt = []
    for k in range(_K):
        c = jnp.sum((rank <= k).astype(jnp.int32), axis=1, keepdims=True)
        cnt.append(c)
    Yc = jnp.concatenate(cnt, axis=1)
    Yc = jnp.where(Yc == _N, Yc[:, 0:1], Yc)
    idx_ref[0, 0] = Yc + b * _N


def _run_ball_query(xyz, nxyz):
    xt = jnp.transpose(xyz, (0, 2, 1))
    nblk = _S // _SB
    idx = pl.pallas_call(
        _bq_body,
        grid=(_B, nblk),
        in_specs=[
            pl.BlockSpec((1, 3, _N), lambda b, j: (b, 0, 0)),
            pl.BlockSpec((1, 1, _SB, 3), lambda b, j: (b, j, 0, 0)),
        ],
        out_specs=pl.BlockSpec((1, 1, _SB, _K), lambda b, j: (b, j, 0, 0)),
        out_shape=jax.ShapeDtypeStruct((_B, nblk, _SB, _K), jnp.int32),
    )(xt, nxyz.reshape(_B, nblk, _SB, 3))
    return idx.reshape(_M)


_NW = 32
_RPW = _M // _NW
_CH = 512


def _sc_gather_body(table_hbm, idx_hbm, out_hbm, idx_v, rows_v, sem):
    wid = lax.axis_index("s") * 2 + lax.axis_index("c")
    base = wid * _RPW

    def chunk(c, carry):
        off = base + c * _CH
        pltpu.sync_copy(idx_hbm.at[pl.ds(off, _CH)], idx_v)
        pltpu.async_copy(table_hbm.at[idx_v], rows_v, sem).wait()
        pltpu.sync_copy(rows_v, out_hbm.at[pl.ds(off, _CH)])
        return carry

    lax.fori_loop(0, _RPW // _CH, chunk, 0)


def _run_gather(table, idx):
    mesh = plsc.VectorSubcoreMesh(core_axis_name="c", subcore_axis_name="s")
    fn = functools.partial(
        pl.kernel,
        mesh=mesh,
        out_type=jax.ShapeDtypeStruct((_M, _DTAB), jnp.float32),
        scratch_types=[
            pltpu.VMEM((_CH,), jnp.int32),
            pltpu.VMEM((_CH, _DTAB), jnp.float32),
            pltpu.SemaphoreType.DMA,
        ],
    )(_sc_gather_body)
    return fn(table, idx)


_RB = 2048


def _mlp0_body(g_ref, c_ref, w_ref, b_ref, y_ref, st_ref, acc):
    i = pl.program_id(0)
    g = g_ref[...]
    cx = c_ref[...]
    dp = (g[:, 0:3] - cx) / _R
    x0 = jnp.concatenate([dp, g[:, 3:_DTAB]], axis=1)
    y = jnp.dot(x0, w_ref[...], preferred_element_type=jnp.float32)
    y = y + b_ref[...]
    s = jnp.sum(y, axis=0, keepdims=True)
    sq = jnp.sum(y * y, axis=0, keepdims=True)
    @pl.when(i == 0)
    def _():
        acc[0:1, :] = s
        acc[1:2, :] = sq
    @pl.when(i > 0)
    def _():
        acc[0:1, :] = acc[0:1, :] + s
        acc[1:2, :] = acc[1:2, :] + sq
    y_ref[...] = y
    @pl.when(i == pl.num_programs(0) - 1)
    def _():
        st_ref[...] = acc[...]


def _mlp_mid_body(y_ref, st_in_ref, ga_ref, be_ref, w_ref, b_ref,
                  y_out_ref, st_ref, acc):
    i = pl.program_id(0)
    mean = st_in_ref[0:1, :] / _M
    var = st_in_ref[1:2, :] / _M - mean * mean
    scale = ga_ref[...] / jnp.sqrt(var + _EPS)
    shift = be_ref[...] - mean * scale
    h = jnp.maximum(y_ref[...] * scale + shift, 0.0)
    y = jnp.dot(h, w_ref[...], preferred_element_type=jnp.float32)
    y = y + b_ref[...]
    s = jnp.sum(y, axis=0, keepdims=True)
    sq = jnp.sum(y * y, axis=0, keepdims=True)
    @pl.when(i == 0)
    def _():
        acc[0:1, :] = s
        acc[1:2, :] = sq
    @pl.when(i > 0)
    def _():
        acc[0:1, :] = acc[0:1, :] + s
        acc[1:2, :] = acc[1:2, :] + sq
    y_out_ref[...] = y
    @pl.when(i == pl.num_programs(0) - 1)
    def _():
        st_ref[...] = acc[...]


def _mlp_final_body(y_ref, st_in_ref, ga_ref, be_ref, out_ref):
    mean = st_in_ref[0:1, :] / _M
    var = st_in_ref[1:2, :] / _M - mean * mean
    scale = ga_ref[...] / jnp.sqrt(var + _EPS)
    shift = be_ref[...] - mean * scale
    h = jnp.maximum(y_ref[...] * scale + shift, 0.0)
    h3 = h.reshape(_RB // _K, _K, h.shape[1])
    out_ref[...] = jnp.max(h3, axis=1)


def _run_mlp(g, cexp, params):
    (w0, b0, g0, be0), (w1, b1, g1, be1), (w2, b2, g2, be2) = params
    nsteps = _M // _RB
    w0p = jnp.zeros((_DTAB, 32), jnp.float32).at[0:35, :].set(w0.T)

    y0, st0 = pl.pallas_call(
        _mlp0_body,
        grid=(nsteps,),
        in_specs=[
            pl.BlockSpec((_RB, _DTAB), lambda i: (i, 0)),
            pl.BlockSpec((_RB, 3), lambda i: (i, 0)),
            pl.BlockSpec((_DTAB, 32), lambda i: (0, 0)),
            pl.BlockSpec((1, 32), lambda i: (0, 0)),
        ],
        out_specs=[
            pl.BlockSpec((_RB, 32), lambda i: (i, 0)),
            pl.BlockSpec((2, 32), lambda i: (0, 0)),
        ],
        out_shape=[
            jax.ShapeDtypeStruct((_M, 32), jnp.float32),
            jax.ShapeDtypeStruct((2, 32), jnp.float32),
        ],
        scratch_shapes=[pltpu.VMEM((2, 32), jnp.float32)],
    )(g, cexp, w0p, b0.reshape(1, 32))

    def mid(y_in, st_in, ga, be, w, b, cout):
        return pl.pallas_call(
            _mlp_mid_body,
            grid=(nsteps,),
            in_specs=[
                pl.BlockSpec((_RB, y_in.shape[1]), lambda i: (i, 0)),
                pl.BlockSpec((2, y_in.shape[1]), lambda i: (0, 0)),
                pl.BlockSpec((1, y_in.shape[1]), lambda i: (0, 0)),
                pl.BlockSpec((1, y_in.shape[1]), lambda i: (0, 0)),
                pl.BlockSpec((y_in.shape[1], cout), lambda i: (0, 0)),
                pl.BlockSpec((1, cout), lambda i: (0, 0)),
            ],
            out_specs=[
                pl.BlockSpec((_RB, cout), lambda i: (i, 0)),
                pl.BlockSpec((2, cout), lambda i: (0, 0)),
            ],
            out_shape=[
                jax.ShapeDtypeStruct((_M, cout), jnp.float32),
                jax.ShapeDtypeStruct((2, cout), jnp.float32),
            ],
            scratch_shapes=[pltpu.VMEM((2, cout), jnp.float32)],
        )(y_in, st_in, ga.reshape(1, -1), be.reshape(1, -1), w.T, b.reshape(1, -1))

    y1, st1 = mid(y0, st0, g0, be0, w1, b1, 32)
    y2, st2 = mid(y1, st1, g1, be1, w2, b2, 64)

    out = pl.pallas_call(
        _mlp_final_body,
        grid=(nsteps,),
        in_specs=[
            pl.BlockSpec((_RB, 64), lambda i: (i, 0)),
            pl.BlockSpec((2, 64), lambda i: (0, 0)),
            pl.BlockSpec((1, 64), lambda i: (0, 0)),
            pl.BlockSpec((1, 64), lambda i: (0, 0)),
        ],
        out_specs=pl.BlockSpec((_RB // _K, 64), lambda i: (i, 0)),
        out_shape=jax.ShapeDtypeStruct((_M // _K, 64), jnp.float32),
    )(y2, st2, g2.reshape(1, 64), be2.reshape(1, 64))
    return out


def kernel(xyz, feats, W0, b0, gamma0, beta0, W1, b1, gamma1, beta1,
           W2, b2, gamma2, beta2):
    cent, new_xyz = _run_fps(xyz)
    idx = _run_ball_query(xyz, new_xyz)
    table = jnp.zeros((_B * _N, _DTAB), jnp.float32)
    table = table.at[:, 0:3].set(xyz.reshape(_B * _N, 3))
    table = table.at[:, 3:3 + _CIN].set(feats.reshape(_B * _N, _CIN))
    g = _run_gather(table, idx)
    cexp = jnp.broadcast_to(new_xyz[:, :, None, :], (_B, _S, _K, 3)).reshape(_M, 3)
    params = [(W0, b0, gamma0, beta0), (W1, b1, gamma1, beta1),
              (W2, b2, gamma2, beta2)]
    new_feats = _run_mlp(g, cexp, params).reshape(_B, _S, 64)
    return (new_xyz, new_feats)
